# baseline (speedup 1.0000x reference)
import functools

import jax
import jax.numpy as jnp
from jax import lax
from jax.experimental import pallas as pl
from jax.experimental.pallas import tpu as pltpu
from jax.experimental.pallas import tpu_sc as plsc

N = 10000
E = 320000
DF = 128
DH = 64
NG = 64
NL = 10

NC, NS, LANES = 2, 16, 16
NW = NC * NS
CHUNK = 128
NBUF = 4
CPT0 = 94
CPT1 = 66
NGRP0 = CPT0 // NBUF
NGRP1 = CPT1 // NBUF
CHTOT = NS * (CPT0 + CPT1)
EPAD = CHTOT * CHUNK
NPAD = 10240
RPT = NPAD // NS
DEGW = 16

_mesh = lambda: plsc.VectorSubcoreMesh(core_axis_name="c", subcore_axis_name="s")


@functools.partial(
    pl.kernel,
    out_type=jax.ShapeDtypeStruct((NC, NPAD, DEGW), jnp.float32),
    mesh=_mesh(),
    compiler_params=pltpu.CompilerParams(use_tc_tiling_on_sc=False),
    scratch_types=[
        pltpu.VMEM((CPT0, CHUNK), jnp.int32),
        pltpu.VMEM((CHUNK, DEGW), jnp.float32),
        pltpu.VMEM_SHARED((NPAD, DEGW), jnp.float32),
        pltpu.SemaphoreType.DMA,
    ],
)
def _sc_degree(dst_hbm, ones_hbm, zeros_hbm, out_hbm, dst_v, ones_v, accum, sem):
    c = lax.axis_index("c")
    s = lax.axis_index("s")
    pltpu.sync_copy(ones_hbm, ones_v)
    pltpu.sync_copy(zeros_hbm.at[pl.ds(s * RPT, RPT)], accum.at[pl.ds(s * RPT, RPT)])
    plsc.subcore_barrier()

    def _run(cptc, base):
        pltpu.sync_copy(dst_hbm.at[pl.ds(base + s * cptc, cptc)],
                        dst_v.at[pl.ds(0, cptc)])

        def fire(j, carry):
            pltpu.async_copy(ones_v, accum.at[dst_v.at[j]], sem, add=True)
            return carry

        lax.fori_loop(0, cptc, fire, 0)

        def drain(j, carry):
            pltpu.make_async_copy(ones_v, accum.at[dst_v.at[j]], sem).wait()
            return carry

        lax.fori_loop(0, cptc, drain, 0)

    @pl.when(c == 0)
    def _():
        _run(CPT0, 0)

    @pl.when(c == 1)
    def _():
        _run(CPT1, NS * CPT0)

    plsc.subcore_barrier()
    pltpu.sync_copy(accum.at[pl.ds(s * RPT, RPT)], out_hbm.at[c, pl.ds(s * RPT, RPT)])


@functools.partial(
    pl.kernel,
    out_type=jax.ShapeDtypeStruct((NC, NPAD, DH), jnp.float32),
    mesh=_mesh(),
    compiler_params=pltpu.CompilerParams(use_tc_tiling_on_sc=False),
    scratch_types=[
        pltpu.VMEM((CPT0, CHUNK), jnp.int32),
        pltpu.VMEM((CPT0, CHUNK), jnp.int32),
        pltpu.VMEM((NBUF, CHUNK, DH), jnp.float32),
        pltpu.VMEM_SHARED((NPAD, DH), jnp.float32),
        pltpu.SemaphoreType.DMA,
        pltpu.SemaphoreType.DMA,
    ],
)
def _sc_agg(g_hbm, src_hbm, dst_hbm, out_hbm, src_v, dst_v, rows_v,
            accum, gsem, ssem):
    c = lax.axis_index("c")
    s = lax.axis_index("s")

    def zrow(r, carry):
        for c4 in range(DH // LANES):
            rows_v[0, r, pl.ds(c4 * LANES, LANES)] = jnp.zeros((LANES,), jnp.float32)
        return carry

    lax.fori_loop(0, CHUNK, zrow, 0)
    for t in range(RPT // CHUNK):
        pltpu.sync_copy(rows_v.at[0], accum.at[pl.ds(s * RPT + t * CHUNK, CHUNK)])
    plsc.subcore_barrier()

    def _run(cptc, ngrpc, base):
        pltpu.sync_copy(src_hbm.at[pl.ds(base + s * cptc, cptc)],
                        src_v.at[pl.ds(0, cptc)])
        pltpu.sync_copy(dst_hbm.at[pl.ds(base + s * cptc, cptc)],
                        dst_v.at[pl.ds(0, cptc)])
        for b in range(NBUF):
            pltpu.async_copy(g_hbm.at[src_v.at[b]], rows_v.at[b], gsem)

        def body(g, carry):
            j0 = g * NBUF
            for b in range(NBUF):
                pltpu.make_async_copy(g_hbm.at[src_v.at[j0 + b]], rows_v.at[b], gsem).wait()
                pltpu.async_copy(rows_v.at[b], accum.at[dst_v.at[j0 + b]], ssem, add=True)
            for b in range(NBUF):
                pltpu.make_async_copy(rows_v.at[b], accum.at[dst_v.at[j0 + b]], ssem).wait()
                jn = jnp.minimum(j0 + NBUF + b, cptc - 1)

                @pl.when(g < ngrpc - 1)
                def _():
                    pltpu.async_copy(g_hbm.at[src_v.at[jn]], rows_v.at[b], gsem)

            return carry

        lax.fori_loop(0, ngrpc, body, 0)

    def _run_serial(cptc, base):
        pltpu.sync_copy(src_hbm.at[pl.ds(base + s * cptc, cptc)],
                        src_v.at[pl.ds(0, cptc)])
        pltpu.sync_copy(dst_hbm.at[pl.ds(base + s * cptc, cptc)],
                        dst_v.at[pl.ds(0, cptc)])

        def body(j, carry):
            pltpu.async_copy(g_hbm.at[src_v.at[j]], rows_v.at[0], gsem).wait()
            pltpu.sync_copy(rows_v.at[0], accum.at[dst_v.at[j]], add=True)
            return carry

        lax.fori_loop(0, cptc, body, 0)

    @pl.when(c == 0)
    def _():
        _run_serial(CPT0, 0)

    @pl.when(c == 1)
    def _():
        _run_serial(CPT1, NS * CPT0)

    plsc.subcore_barrier()
    pltpu.sync_copy(accum.at[pl.ds(s * RPT, RPT)], out_hbm.at[c, pl.ds(s * RPT, RPT)])


_BR = 2000
_NB = N // _BR


def _tc1_body(x_ref, w1_ref, dp_ref, g1_ref, dinv_ref):
    deg = dp_ref[0, :, 0:1] + dp_ref[1, :, 0:1] + 1.0
    dinv = lax.rsqrt(deg)
    h = jnp.dot(x_ref[...], w1_ref[...], preferred_element_type=jnp.float32)
    g1_ref[...] = h * dinv
    dinv_ref[...] = dinv


def _tc1(x, W1, dp):
    return pl.pallas_call(
        _tc1_body,
        grid=(_NB,),
        in_specs=[
            pl.BlockSpec((_BR, DF), lambda i: (i, 0)),
            pl.BlockSpec((DF, DH), lambda i: (0, 0)),
            pl.BlockSpec((NC, _BR, DEGW), lambda i: (0, i, 0)),
        ],
        out_specs=[
            pl.BlockSpec((_BR, DH), lambda i: (i, 0)),
            pl.BlockSpec((_BR, 1), lambda i: (i, 0)),
        ],
        out_shape=[
            jax.ShapeDtypeStruct((N, DH), jnp.float32),
            jax.ShapeDtypeStruct((N, 1), jnp.float32),
        ],
    )(x, W1, dp)


def _tc2_body(sp_ref, g1_ref, dinv_ref, b1_ref, w2_ref, g2_ref):
    dinv = dinv_ref[...]
    stot = sp_ref[0] + sp_ref[1] + g1_ref[...]
    h = jnp.maximum(dinv * stot + b1_ref[...], 0.0)
    g2_ref[...] = jnp.dot(h, w2_ref[...], preferred_element_type=jnp.float32) * dinv


def _tc2(sp, g1, dinv, b1, W2):
    return pl.pallas_call(
        _tc2_body,
        grid=(_NB,),
        in_specs=[
            pl.BlockSpec((NC, _BR, DH), lambda i: (0, i, 0)),
            pl.BlockSpec((_BR, DH), lambda i: (i, 0)),
            pl.BlockSpec((_BR, 1), lambda i: (i, 0)),
            pl.BlockSpec((1, DH), lambda i: (0, 0)),
            pl.BlockSpec((DH, DH), lambda i: (0, 0)),
        ],
        out_specs=pl.BlockSpec((_BR, DH), lambda i: (i, 0)),
        out_shape=jax.ShapeDtypeStruct((N, DH), jnp.float32),
    )(sp, g1, dinv, b1, W2)


def _tc3_body(sp_ref, g2_ref, dinv_ref, batch_ref, b2_ref, wf1_ref, bf1_ref,
              wf2_ref, bf2_ref, emb_ref, pred_ref, sums_acc, cnts_acc):
    i = pl.program_id(0)

    @pl.when(i == 0)
    def _():
        sums_acc[...] = jnp.zeros_like(sums_acc)
        cnts_acc[...] = jnp.zeros_like(cnts_acc)

    dinv = dinv_ref[...]
    stot = sp_ref[0] + sp_ref[1] + g2_ref[...]
    h2 = jnp.maximum(dinv * stot + b2_ref[...], 0.0)
    a1 = jnp.dot(h2, wf1_ref[...], preferred_element_type=jnp.float32) + bf1_ref[...]
    gid = lax.broadcasted_iota(jnp.int32, (_BR, NG), 1)
    oh = (batch_ref[...] == gid).astype(jnp.float32)
    dn = (((0,), (0,)), ((), ()))
    sums_acc[...] += lax.dot_general(oh, a1, dn, preferred_element_type=jnp.float32)
    cnts_acc[...] += lax.dot_general(oh, jnp.ones_like(a1), dn,
                                     preferred_element_type=jnp.float32)

    @pl.when(i == _NB - 1)
    def _():
        emb = sums_acc[...] / jnp.maximum(cnts_acc[...], 1.0)
        emb_ref[...] = emb
        pred_ref[...] = jnp.dot(emb, wf2_ref[...],
                                preferred_element_type=jnp.float32) + bf2_ref[...]


def _tc3(sp, g2, dinv, batch2, b2, Wf1, bf1, Wf2, bf2):
    return pl.pallas_call(
        _tc3_body,
        grid=(_NB,),
        in_specs=[
            pl.BlockSpec((NC, _BR, DH), lambda i: (0, i, 0)),
            pl.BlockSpec((_BR, DH), lambda i: (i, 0)),
            pl.BlockSpec((_BR, 1), lambda i: (i, 0)),
            pl.BlockSpec((_BR, 1), lambda i: (i, 0)),
            pl.BlockSpec((1, DH), lambda i: (0, 0)),
            pl.BlockSpec((DH, DH), lambda i: (0, 0)),
            pl.BlockSpec((1, DH), lambda i: (0, 0)),
            pl.BlockSpec((DH, NL), lambda i: (0, 0)),
            pl.BlockSpec((1, NL), lambda i: (0, 0)),
        ],
        out_specs=[
            pl.BlockSpec((NG, DH), lambda i: (0, 0)),
            pl.BlockSpec((NG, NL), lambda i: (0, 0)),
        ],
        out_shape=[
            jax.ShapeDtypeStruct((NG, DH), jnp.float32),
            jax.ShapeDtypeStruct((NG, NL), jnp.float32),
        ],
        scratch_shapes=[
            pltpu.VMEM((NG, DH), jnp.float32),
            pltpu.VMEM((NG, DH), jnp.float32),
        ],
    )(sp, g2, dinv, batch2, b2, Wf1, bf1, Wf2, bf2)


def kernel(x, edge_index, batch, W1, b1, W2, b2, Wf1, bf1, Wf2, bf2):
    src = edge_index[0]
    dst = edge_index[1]
    pad = EPAD - E
    src_p = jnp.concatenate([src, jnp.zeros((pad,), jnp.int32)]).reshape(CHTOT, CHUNK)
    dst_p = jnp.concatenate([dst, jnp.full((pad,), NPAD - 1, jnp.int32)]).reshape(CHTOT, CHUNK)
    ones16 = jnp.ones((CHUNK, DEGW), jnp.float32)
    zeros16 = jnp.zeros((NPAD, DEGW), jnp.float32)

    dp = _sc_degree(dst_p, ones16, zeros16)
    g1, dinv = _tc1(x, W1, dp)
    sp1 = _sc_agg(g1, src_p, dst_p)
    g2 = _tc2(sp1, g1, dinv, b1.reshape(1, -1), W2)
    sp2 = _sc_agg(g2, src_p, dst_p)
    emb, pred = _tc3(sp2, g2, dinv, batch.reshape(-1, 1),
                     b2.reshape(1, -1), Wf1, bf1.reshape(1, -1),
                     Wf2, bf2.reshape(1, -1))
    return emb, jnp.asarray(0.0, jnp.float32), pred


# speedup vs baseline: 1.3122x; 1.3122x over previous
import functools

import jax
import jax.numpy as jnp
from jax import lax
from jax.experimental import pallas as pl
from jax.experimental.pallas import tpu as pltpu
from jax.experimental.pallas import tpu_sc as plsc

N = 10000
E = 320000
DF = 128
DH = 64
NG = 64
NL = 10

NC, NS, LANES = 2, 16, 16
NW = NC * NS
CHUNK = 128
CPT = -(-E // (NW * CHUNK))
EPAD = NW * CPT * CHUNK
NPAD = 10240
RPT = NPAD // NS

_mesh = lambda: plsc.VectorSubcoreMesh(core_axis_name="c", subcore_axis_name="s")


@functools.partial(
    pl.kernel,
    out_type=jax.ShapeDtypeStruct((NC, NPAD, LANES), jnp.float32),
    mesh=_mesh(),
    compiler_params=pltpu.CompilerParams(use_tc_tiling_on_sc=False),
    scratch_types=[
        pltpu.VMEM((CPT, CHUNK), jnp.int32),
        pltpu.VMEM((CHUNK, LANES), jnp.float32),
        pltpu.VMEM_SHARED((NPAD, LANES), jnp.float32),
    ],
)
def _sc_degree(dst_hbm, ones_hbm, zeros_hbm, out_hbm, dst_v, ones_v, accum):
    c = lax.axis_index("c")
    s = lax.axis_index("s")
    wid = s * NC + c
    pltpu.sync_copy(dst_hbm.at[wid], dst_v)
    pltpu.sync_copy(ones_hbm, ones_v)
    pltpu.sync_copy(zeros_hbm.at[pl.ds(s * RPT, RPT)], accum.at[pl.ds(s * RPT, RPT)])
    plsc.subcore_barrier()

    def body(j, carry):
        pltpu.sync_copy(ones_v, accum.at[dst_v.at[j]], add=True)
        return carry

    lax.fori_loop(0, CPT, body, 0)
    plsc.subcore_barrier()
    pltpu.sync_copy(accum.at[pl.ds(s * RPT, RPT)], out_hbm.at[c, pl.ds(s * RPT, RPT)])


@functools.partial(
    pl.kernel,
    out_type=jax.ShapeDtypeStruct((NC, NPAD, DH), jnp.float32),
    mesh=_mesh(),
    compiler_params=pltpu.CompilerParams(use_tc_tiling_on_sc=False),
    scratch_types=[
        pltpu.VMEM((CPT, CHUNK), jnp.int32),
        pltpu.VMEM((CPT, CHUNK), jnp.int32),
        pltpu.VMEM((CHUNK, DH), jnp.float32),
        pltpu.VMEM_SHARED((NPAD, DH), jnp.float32),
        pltpu.SemaphoreType.DMA,
    ],
)
def _sc_agg(g_hbm, src_hbm, dst_hbm, zeros_hbm, out_hbm, src_v, dst_v, rows_v, accum, sem):
    c = lax.axis_index("c")
    s = lax.axis_index("s")
    wid = s * NC + c
    pltpu.sync_copy(src_hbm.at[wid], src_v)
    pltpu.sync_copy(dst_hbm.at[wid], dst_v)
    pltpu.sync_copy(zeros_hbm.at[pl.ds(s * RPT, RPT)], accum.at[pl.ds(s * RPT, RPT)])
    plsc.subcore_barrier()

    def body(j, carry):
        pltpu.async_copy(g_hbm.at[src_v.at[j]], rows_v, sem).wait()
        pltpu.sync_copy(rows_v, accum.at[dst_v.at[j]], add=True)
        return carry

    lax.fori_loop(0, CPT, body, 0)
    plsc.subcore_barrier()
    pltpu.sync_copy(accum.at[pl.ds(s * RPT, RPT)], out_hbm.at[c, pl.ds(s * RPT, RPT)])


_BR = 2000
_NB = N // _BR


def _tc1_body(x_ref, w1_ref, dp_ref, g1_ref, dinv_ref):
    deg = dp_ref[0, :, 0:1] + dp_ref[1, :, 0:1] + 1.0
    dinv = lax.rsqrt(deg)
    h = jnp.dot(x_ref[...], w1_ref[...], preferred_element_type=jnp.float32)
    g1_ref[...] = h * dinv
    dinv_ref[...] = dinv


def _tc1(x, W1, dp):
    return pl.pallas_call(
        _tc1_body,
        grid=(_NB,),
        in_specs=[
            pl.BlockSpec((_BR, DF), lambda i: (i, 0)),
            pl.BlockSpec((DF, DH), lambda i: (0, 0)),
            pl.BlockSpec((NC, _BR, LANES), lambda i: (0, i, 0)),
        ],
        out_specs=[
            pl.BlockSpec((_BR, DH), lambda i: (i, 0)),
            pl.BlockSpec((_BR, 1), lambda i: (i, 0)),
        ],
        out_shape=[
            jax.ShapeDtypeStruct((N, DH), jnp.float32),
            jax.ShapeDtypeStruct((N, 1), jnp.float32),
        ],
    )(x, W1, dp)


def _tc2_body(sp_ref, g1_ref, dinv_ref, b1_ref, w2_ref, g2_ref):
    dinv = dinv_ref[...]
    stot = sp_ref[0] + sp_ref[1] + g1_ref[...]
    h = jnp.maximum(dinv * stot + b1_ref[...], 0.0)
    g2_ref[...] = jnp.dot(h, w2_ref[...], preferred_element_type=jnp.float32) * dinv


def _tc2(sp, g1, dinv, b1, W2):
    return pl.pallas_call(
        _tc2_body,
        grid=(_NB,),
        in_specs=[
            pl.BlockSpec((NC, _BR, DH), lambda i: (0, i, 0)),
            pl.BlockSpec((_BR, DH), lambda i: (i, 0)),
            pl.BlockSpec((_BR, 1), lambda i: (i, 0)),
            pl.BlockSpec((1, DH), lambda i: (0, 0)),
            pl.BlockSpec((DH, DH), lambda i: (0, 0)),
        ],
        out_specs=pl.BlockSpec((_BR, DH), lambda i: (i, 0)),
        out_shape=jax.ShapeDtypeStruct((N, DH), jnp.float32),
    )(sp, g1, dinv, b1, W2)


def _tc3_body(sp_ref, g2_ref, dinv_ref, batch_ref, b2_ref, wf1_ref, bf1_ref,
              wf2_ref, bf2_ref, emb_ref, pred_ref, sums_acc, cnts_acc):
    i = pl.program_id(0)

    @pl.when(i == 0)
    def _():
        sums_acc[...] = jnp.zeros_like(sums_acc)
        cnts_acc[...] = jnp.zeros_like(cnts_acc)

    dinv = dinv_ref[...]
    stot = sp_ref[0] + sp_ref[1] + g2_ref[...]
    h2 = jnp.maximum(dinv * stot + b2_ref[...], 0.0)
    a1 = jnp.dot(h2, wf1_ref[...], preferred_element_type=jnp.float32) + bf1_ref[...]
    gid = lax.broadcasted_iota(jnp.int32, (_BR, NG), 1)
    oh = (batch_ref[...] == gid).astype(jnp.float32)
    dn = (((0,), (0,)), ((), ()))
    sums_acc[...] += lax.dot_general(oh, a1, dn, preferred_element_type=jnp.float32)
    cnts_acc[...] += lax.dot_general(oh, jnp.ones_like(a1), dn,
                                     preferred_element_type=jnp.float32)

    @pl.when(i == _NB - 1)
    def _():
        emb = sums_acc[...] / jnp.maximum(cnts_acc[...], 1.0)
        emb_ref[...] = emb
        pred_ref[...] = jnp.dot(emb, wf2_ref[...],
                                preferred_element_type=jnp.float32) + bf2_ref[...]


def _tc3(sp, g2, dinv, batch2, b2, Wf1, bf1, Wf2, bf2):
    return pl.pallas_call(
        _tc3_body,
        grid=(_NB,),
        in_specs=[
            pl.BlockSpec((NC, _BR, DH), lambda i: (0, i, 0)),
            pl.BlockSpec((_BR, DH), lambda i: (i, 0)),
            pl.BlockSpec((_BR, 1), lambda i: (i, 0)),
            pl.BlockSpec((_BR, 1), lambda i: (i, 0)),
            pl.BlockSpec((1, DH), lambda i: (0, 0)),
            pl.BlockSpec((DH, DH), lambda i: (0, 0)),
            pl.BlockSpec((1, DH), lambda i: (0, 0)),
            pl.BlockSpec((DH, NL), lambda i: (0, 0)),
            pl.BlockSpec((1, NL), lambda i: (0, 0)),
        ],
        out_specs=[
            pl.BlockSpec((NG, DH), lambda i: (0, 0)),
            pl.BlockSpec((NG, NL), lambda i: (0, 0)),
        ],
        out_shape=[
            jax.ShapeDtypeStruct((NG, DH), jnp.float32),
            jax.ShapeDtypeStruct((NG, NL), jnp.float32),
        ],
        scratch_shapes=[
            pltpu.VMEM((NG, DH), jnp.float32),
            pltpu.VMEM((NG, DH), jnp.float32),
        ],
    )(sp, g2, dinv, batch2, b2, Wf1, bf1, Wf2, bf2)


def kernel(x, edge_index, batch, W1, b1, W2, b2, Wf1, bf1, Wf2, bf2):
    src = edge_index[0]
    dst = edge_index[1]
    pad = EPAD - E
    src_p = jnp.concatenate([src, jnp.zeros((pad,), jnp.int32)]).reshape(NW, CPT, CHUNK)
    dst_p = jnp.concatenate([dst, jnp.full((pad,), NPAD - 1, jnp.int32)]).reshape(NW, CPT, CHUNK)
    ones16 = jnp.ones((CHUNK, LANES), jnp.float32)
    zeros16 = jnp.zeros((NPAD, LANES), jnp.float32)
    zeros64 = jnp.zeros((NPAD, DH), jnp.float32)

    dp = _sc_degree(dst_p, ones16, zeros16)
    g1, dinv = _tc1(x, W1, dp)
    sp1 = _sc_agg(g1, src_p, dst_p, zeros64)
    g2 = _tc2(sp1, g1, dinv, b1.reshape(1, -1), W2)
    sp2 = _sc_agg(g2, src_p, dst_p, zeros64)
    emb, pred = _tc3(sp2, g2, dinv, batch.reshape(-1, 1),
                     b2.reshape(1, -1), Wf1, bf1.reshape(1, -1),
                     Wf2, bf2.reshape(1, -1))
    return emb, jnp.asarray(0.0, jnp.float32), pred
